# baseline (speedup 1.0000x reference)
import functools

import jax
import jax.numpy as jnp
from jax import lax
from jax.experimental import pallas as pl
from jax.experimental.pallas import tpu as pltpu
from jax.experimental.pallas import tpu_sc as plsc

CH = 128


@functools.lru_cache(maxsize=None)
def _make_gather(V, D, B, L):
    info = plsc.get_sparse_core_info()
    NC, NS, NL = info.num_cores, info.num_subcores, info.num_lanes
    NW = NC * NS
    N = B * L
    assert D == 64 and V % 2 == 0 and B % CH == 0 and N % (NW * CH) == 0
    n_blocks = N // (NW * CH)
    blocks_per_l = B // CH
    assert n_blocks % 2 == 0
    NG = CH // NL

    mesh = plsc.VectorSubcoreMesh(core_axis_name="c", subcore_axis_name="s")

    @functools.partial(
        pl.kernel,
        mesh=mesh,
        compiler_params=pltpu.CompilerParams(
            use_tc_tiling_on_sc=True, needs_layout_passes=False
        ),
        out_type=jax.ShapeDtypeStruct((L, D, B), jnp.float32),
        scratch_types=[
            pltpu.VMEM((n_blocks, CH), jnp.int32),
            pltpu.VMEM((2, CH), jnp.int32),
            pltpu.VMEM((2, CH, 2 * D), jnp.float32),
            pltpu.VMEM((2, D, CH), jnp.float32),
            [pltpu.SemaphoreType.DMA] * 2,
            [pltpu.SemaphoreType.DMA] * 2,
        ],
    )
    def k(x_hbm, t2_hbm, out_hbm, idx_v, pair_v, gbuf, tbuf, gsems, osems):
        wid = lax.axis_index("s") * NC + lax.axis_index("c")
        gbase = wid * n_blocks
        pltpu.sync_copy(x_hbm.at[wid], idx_v)
        srows = [lax.iota(jnp.int32, NL) + g * NL for g in range(NG)]

        def prep_and_fire(j, p):
            for g in range(NG):
                iv = idx_v[j, pl.ds(g * NL, NL)]
                pair_v[p, pl.ds(g * NL, NL)] = lax.shift_right_logical(iv, 1)
            pltpu.async_copy(t2_hbm.at[pair_v.at[p]], gbuf.at[p], gsems[p])

        def out_slice(l, b0):
            return out_hbm.at[l, :, pl.ds(b0, CH)]

        for p in range(2):
            prep_and_fire(p, p)

        def body(gg, _):
            for p in range(2):
                j = 2 * gg + p
                G = gbase + j
                l = G // blocks_per_l
                b0 = (G % blocks_per_l) * CH
                pltpu.make_async_copy(
                    t2_hbm.at[pair_v.at[p]], gbuf.at[p], gsems[p]
                ).wait()

                @pl.when(j >= 2)
                def _():
                    pltpu.make_async_copy(
                        tbuf.at[p], out_slice(l, b0), osems[p]
                    ).wait()

                colbs = [
                    lax.shift_left(
                        lax.bitwise_and(idx_v[j, pl.ds(g * NL, NL)], 1), 6
                    )
                    for g in range(NG)
                ]

                def dbody(d, _):
                    vs = [
                        plsc.load_gather(gbuf.at[p], [srows[g], colbs[g] + d])
                        for g in range(NG)
                    ]
                    for g in range(NG):
                        tbuf[p, d, pl.ds(g * NL, NL)] = vs[g]
                    return 0

                lax.fori_loop(0, D, dbody, 0, unroll=8)
                pltpu.async_copy(tbuf.at[p], out_slice(l, b0), osems[p])

                @pl.when(j + 2 < n_blocks)
                def _():
                    prep_and_fire(j + 2, p)

            return 0

        lax.fori_loop(0, n_blocks // 2, body, 0, unroll=False)
        for p in range(2):
            G = gbase + n_blocks - 2 + p
            pltpu.make_async_copy(
                tbuf.at[p],
                out_slice(G // blocks_per_l, (G % blocks_per_l) * CH),
                osems[p],
            ).wait()

    return k


def kernel(x, table):
    B, L = x.shape
    V, D = table.shape
    info = plsc.get_sparse_core_info()
    NW = info.num_cores * info.num_subcores
    xt = x.T.astype(jnp.int32).reshape(NW, (B * L) // (NW * CH), CH)
    t2 = jnp.concatenate([table[0::2], table[1::2]], axis=1)
    out_p = _make_gather(V, D, B, L)(xt, t2)
    return jnp.transpose(out_p, (2, 0, 1))


# speedup vs baseline: 6.1250x; 6.1250x over previous
import functools

import jax
import jax.numpy as jnp
from jax import lax
from jax.experimental import pallas as pl
from jax.experimental.pallas import tpu as pltpu
from jax.experimental.pallas import tpu_sc as plsc

CH = 128


@functools.lru_cache(maxsize=None)
def _make_gather(V, D, B, L):
    info = plsc.get_sparse_core_info()
    NC, NS, NL = info.num_cores, info.num_subcores, info.num_lanes
    NW = NC * NS
    N = B * L
    assert D == 64 and V % 2 == 0 and B % CH == 0 and N % (NW * CH) == 0
    n_blocks = N // (NW * CH)
    blocks_per_l = B // CH
    assert n_blocks % 2 == 0
    NG = CH // NL

    mesh = plsc.VectorSubcoreMesh(core_axis_name="c", subcore_axis_name="s")

    @functools.partial(
        pl.kernel,
        mesh=mesh,
        compiler_params=pltpu.CompilerParams(
            use_tc_tiling_on_sc=True, needs_layout_passes=False
        ),
        out_type=jax.ShapeDtypeStruct((L, D, B), jnp.float32),
        scratch_types=[
            pltpu.VMEM((n_blocks, CH), jnp.int32),
            pltpu.VMEM((2, CH), jnp.int32),
            pltpu.VMEM((2, CH, 2 * D), jnp.float32),
            pltpu.VMEM((2, D, CH), jnp.float32),
            [pltpu.SemaphoreType.DMA] * 2,
            [pltpu.SemaphoreType.DMA] * 2,
        ],
    )
    def k(x_hbm, t2_hbm, out_hbm, idx_v, pair_v, gbuf, tbuf, gsems, osems):
        wid = lax.axis_index("s") * NC + lax.axis_index("c")
        gbase = wid * n_blocks
        pltpu.sync_copy(x_hbm.at[wid], idx_v)
        srows = [lax.iota(jnp.int32, NL) + g * NL for g in range(NG)]

        def prep_and_fire(j, p):
            for g in range(NG):
                iv = idx_v[j, pl.ds(g * NL, NL)]
                pair_v[p, pl.ds(g * NL, NL)] = lax.shift_right_logical(iv, 1)
            pltpu.async_copy(t2_hbm.at[pair_v.at[p]], gbuf.at[p], gsems[p])

        def out_slice(l, b0):
            return out_hbm.at[l, :, pl.ds(b0, CH)]

        for p in range(2):
            prep_and_fire(p, p)

        def body(gg, _):
            for p in range(2):
                j = 2 * gg + p
                G = gbase + j
                l = G // blocks_per_l
                b0 = (G % blocks_per_l) * CH
                pltpu.make_async_copy(
                    t2_hbm.at[pair_v.at[p]], gbuf.at[p], gsems[p]
                ).wait()

                @pl.when(j >= 2)
                def _():
                    pltpu.make_async_copy(
                        tbuf.at[p], out_slice(l, b0), osems[p]
                    ).wait()

                colbs = [
                    lax.shift_left(
                        lax.bitwise_and(idx_v[j, pl.ds(g * NL, NL)], 1), 6
                    )
                    for g in range(NG)
                ]

                def dbody(d, _):
                    vs = [
                        plsc.load_gather(gbuf.at[p], [srows[g], colbs[g] + d])
                        for g in range(NG)
                    ]
                    for g in range(NG):
                        tbuf[p, d, pl.ds(g * NL, NL)] = vs[g]
                    return 0

                lax.fori_loop(0, D, dbody, 0, unroll=8)
                pltpu.async_copy(tbuf.at[p], out_slice(l, b0), osems[p])

                @pl.when(j + 2 < n_blocks)
                def _():
                    prep_and_fire(j + 2, p)

            return 0

        lax.fori_loop(0, n_blocks // 2, body, 0, unroll=False)
        for p in range(2):
            G = gbase + n_blocks - 2 + p
            pltpu.make_async_copy(
                tbuf.at[p],
                out_slice(G // blocks_per_l, (G % blocks_per_l) * CH),
                osems[p],
            ).wait()

    return k


def kernel(x, table):
    B, L = x.shape
    V, D = table.shape
    info = plsc.get_sparse_core_info()
    NW = info.num_cores * info.num_subcores
    xt = x.T.astype(jnp.int32).reshape(NW, (B * L) // (NW * CH), CH)
    t2 = table.reshape(V // 2, 2 * D)
    out_p = _make_gather(V, D, B, L)(xt, t2)
    return jnp.transpose(out_p, (2, 0, 1))


# speedup vs baseline: 7.9392x; 1.2962x over previous
import functools

import jax
import jax.numpy as jnp
from jax import lax
from jax.experimental import pallas as pl
from jax.experimental.pallas import tpu as pltpu
from jax.experimental.pallas import tpu_sc as plsc

CH = 128


@functools.lru_cache(maxsize=None)
def _make_gather(V, D, N):
    info = plsc.get_sparse_core_info()
    NC, NS = info.num_cores, info.num_subcores
    NW = NC * NS
    assert N % (NW * CH) == 0
    n_per_w = N // NW
    n_chunks = n_per_w // CH

    mesh = plsc.VectorSubcoreMesh(core_axis_name="c", subcore_axis_name="s")

    NBUF = 8
    assert n_chunks % NBUF == 0
    n_groups = n_chunks // NBUF

    @functools.partial(
        pl.kernel,
        mesh=mesh,
        compiler_params=pltpu.CompilerParams(use_tc_tiling_on_sc=False),
        out_type=jax.ShapeDtypeStruct((N, D), jnp.float32),
        scratch_types=[
            pltpu.VMEM((n_chunks, CH), jnp.int32),
            pltpu.VMEM((NBUF, CH, D), jnp.float32),
            [pltpu.SemaphoreType.DMA] * NBUF,
            [pltpu.SemaphoreType.DMA] * NBUF,
        ],
    )
    def k(x_hbm, table_hbm, out_hbm, idx_v, rows_v, gsems, osems):
        wid = lax.axis_index("s") * NC + lax.axis_index("c")
        base = wid * n_per_w
        pltpu.sync_copy(x_hbm.at[wid], idx_v)
        for b in range(NBUF):
            pltpu.async_copy(table_hbm.at[idx_v.at[b]], rows_v.at[b], gsems[b])

        def body(g, _):
            j0 = g * NBUF
            for b in range(NBUF):
                pltpu.make_async_copy(
                    table_hbm.at[idx_v.at[j0 + b]], rows_v.at[b], gsems[b]
                ).wait()
                pltpu.async_copy(
                    rows_v.at[b],
                    out_hbm.at[pl.ds(base + (j0 + b) * CH, CH)],
                    osems[b],
                )
            for b in range(NBUF):
                pltpu.make_async_copy(
                    rows_v.at[b],
                    out_hbm.at[pl.ds(base + (j0 + b) * CH, CH)],
                    osems[b],
                ).wait()

                @pl.when(g + 1 < n_groups)
                def _():
                    pltpu.async_copy(
                        table_hbm.at[idx_v.at[j0 + NBUF + b]],
                        rows_v.at[b],
                        gsems[b],
                    )

            return 0

        lax.fori_loop(0, n_groups, body, 0, unroll=False)

    return k


def kernel(x, table):
    B, L = x.shape
    V, D = table.shape
    N = B * L
    info = plsc.get_sparse_core_info()
    NW = info.num_cores * info.num_subcores
    x_flat = x.reshape(NW, N // (NW * CH), CH).astype(jnp.int32)
    out = _make_gather(V, D, N)(x_flat, table)
    return out.reshape(B, L, D)
